# baseline (speedup 1.0000x reference)
import jax
import jax.numpy as jnp
from jax import lax
from jax.experimental import pallas as pl
from jax.experimental.pallas import tpu as pltpu
from jax.experimental.pallas import tpu_sc as plsc

_N = 10000
_E = 320000
_NC, _NS = 2, 16
_NW = _NC * _NS
_K = 128
_CPT = 80
_G = 16
_NG = _CPT // _G
_EPAD = _NW * _CPT * _K
_NPAD = 10112
_RPT = _NPAD // _NS
_CW = 16
_BLK = 1000
_GRID = _N // _BLK

_f32 = jnp.float32
_sc_params = pltpu.CompilerParams(use_tc_tiling_on_sc=False)


def _core_outs(c, sl, srcs, outs_a, outs_b):
    @pl.when(c == 0)
    def _():
        for src, out in zip(srcs, outs_a):
            pltpu.sync_copy(src.at[sl], out.at[sl])

    @pl.when(c == 1)
    def _():
        for src, out in zip(srcs, outs_b):
            pltpu.sync_copy(src.at[sl], out.at[sl])


def _make_sc_segsum(width):
    mesh = plsc.VectorSubcoreMesh(core_axis_name="c", subcore_axis_name="s",
                                  num_cores=_NC, num_subcores=_NS)
    out_type = [jax.ShapeDtypeStruct((_NPAD, width), _f32)] * 2
    scratch = [pltpu.VMEM_SHARED((_NPAD, width), _f32),
               pltpu.VMEM((_G, _K), jnp.int32),
               pltpu.VMEM((_G, _K), jnp.int32),
               pltpu.VMEM((2, _K, width), _f32),
               pltpu.SemaphoreType.DMA,
               pltpu.SemaphoreType.DMA]

    def body(table, src_hbm, dst_hbm, zeros_hbm, out_a, out_b,
             acc, src_v, dst_v, rows, sem, sem_s):
        c = lax.axis_index("c")
        s = lax.axis_index("s")
        wid = s * _NC + c
        sl = pl.ds(s * _RPT, _RPT)
        pltpu.sync_copy(zeros_hbm, acc.at[sl])
        plsc.subcore_barrier()

        def group(g, carry):
            pltpu.sync_copy(src_hbm.at[wid, pl.ds(g * _G, _G)], src_v)
            pltpu.sync_copy(dst_hbm.at[wid, pl.ds(g * _G, _G)], dst_v)
            pltpu.async_copy(table.at[src_v.at[0]], rows.at[0], sem)

            def chunk(j, carry2):
                b = lax.rem(j, 2)
                pltpu.make_async_copy(
                    table.at[src_v.at[j]], rows.at[b], sem).wait()

                @pl.when(j > 0)
                def _():
                    pltpu.make_async_copy(
                        rows.at[1 - b], acc.at[dst_v.at[j - 1]],
                        sem_s).wait()

                @pl.when(j + 1 < _G)
                def _():
                    pltpu.async_copy(
                        table.at[src_v.at[j + 1]], rows.at[1 - b], sem)

                pltpu.async_copy(rows.at[b], acc.at[dst_v.at[j]], sem_s,
                                 add=True)
                return carry2

            lax.fori_loop(0, _G, chunk, 0)
            pltpu.make_async_copy(
                rows.at[(_G - 1) % 2], acc.at[dst_v.at[_G - 1]],
                sem_s).wait()
            return carry

        lax.fori_loop(0, _NG, group, 0)
        plsc.subcore_barrier()
        _core_outs(c, sl, [acc], [out_a], [out_b])

    return pl.kernel(body, out_type, mesh=mesh, scratch_types=scratch,
                     compiler_params=_sc_params)


def _make_sc_count():
    mesh = plsc.VectorSubcoreMesh(core_axis_name="c", subcore_axis_name="s",
                                  num_cores=_NC, num_subcores=_NS)
    out_type = [jax.ShapeDtypeStruct((_NPAD, _CW), _f32)] * 2
    scratch = [pltpu.VMEM_SHARED((_NPAD, _CW), _f32),
               pltpu.VMEM((_G, _K), jnp.int32),
               pltpu.VMEM((_K, _CW), _f32),
               pltpu.SemaphoreType.DMA]

    def body(dst_hbm, zc_hbm, ones_hbm, cout_a, cout_b,
             cacc, dst_v, ones_v, sem):
        c = lax.axis_index("c")
        s = lax.axis_index("s")
        wid = s * _NC + c
        sl = pl.ds(s * _RPT, _RPT)
        pltpu.sync_copy(zc_hbm, cacc.at[sl])
        pltpu.sync_copy(ones_hbm, ones_v)
        plsc.subcore_barrier()

        def group(g, carry):
            pltpu.sync_copy(dst_hbm.at[wid, pl.ds(g * _G, _G)], dst_v)

            def chunk(j, carry2):
                pltpu.async_copy(ones_v, cacc.at[dst_v.at[j]], sem, add=True)

                @pl.when(j > 0)
                def _():
                    pltpu.make_async_copy(
                        ones_v, cacc.at[dst_v.at[j - 1]], sem).wait()

                return carry2

            lax.fori_loop(0, _G, chunk, 0)
            pltpu.make_async_copy(ones_v, cacc.at[dst_v.at[_G - 1]],
                                  sem).wait()
            return carry

        lax.fori_loop(0, _NG, group, 0)
        plsc.subcore_barrier()
        _core_outs(c, sl, [cacc], [cout_a], [cout_b])

    return pl.kernel(body, out_type, mesh=mesh, scratch_types=scratch,
                     compiler_params=_sc_params)


_sc128 = _make_sc_segsum(128)
_sc64 = _make_sc_segsum(64)
_sc_cnt = _make_sc_count()


def _tc_first(x, wlT, wrT):
    def body(x_r, wl_r, wr_r, y_r, r_r):
        xb = x_r[...]
        y_r[...] = jnp.dot(xb, wl_r[...], preferred_element_type=_f32)
        r_r[...] = jnp.dot(xb, wr_r[...], preferred_element_type=_f32)

    return pl.pallas_call(
        body, grid=(_GRID,),
        in_specs=[pl.BlockSpec((_BLK, 128), lambda i: (i, 0)),
                  pl.BlockSpec((128, 128), lambda i: (0, 0)),
                  pl.BlockSpec((128, 128), lambda i: (0, 0))],
        out_specs=[pl.BlockSpec((_BLK, 128), lambda i: (i, 0))] * 2,
        out_shape=[jax.ShapeDtypeStruct((_N, 128), _f32)] * 2,
    )(x, wlT, wrT)


def _tc_mid(agg_a, agg_b, cnt_a, cnt_b, r, bl, wlT, wrT):
    win = agg_a.shape[1]
    wout = wlT.shape[1]

    def body(aA, aB, cA, cB, r_r, bl_r, wl_r, wr_r, y_r, rr_r):
        cnt = cA[...][:, :1] + cB[...][:, :1]
        inv = 1.0 / jnp.maximum(cnt, 1.0)
        h = jnp.maximum((aA[...] + aB[...]) * inv + bl_r[...] + r_r[...], 0.0)
        y_r[...] = jnp.dot(h, wl_r[...], preferred_element_type=_f32)
        rr_r[...] = jnp.dot(h, wr_r[...], preferred_element_type=_f32)

    return pl.pallas_call(
        body, grid=(_GRID,),
        in_specs=[pl.BlockSpec((_BLK, win), lambda i: (i, 0)),
                  pl.BlockSpec((_BLK, win), lambda i: (i, 0)),
                  pl.BlockSpec((_BLK, _CW), lambda i: (i, 0)),
                  pl.BlockSpec((_BLK, _CW), lambda i: (i, 0)),
                  pl.BlockSpec((_BLK, win), lambda i: (i, 0)),
                  pl.BlockSpec((1, win), lambda i: (0, 0)),
                  pl.BlockSpec((win, wout), lambda i: (0, 0)),
                  pl.BlockSpec((win, wout), lambda i: (0, 0))],
        out_specs=[pl.BlockSpec((_BLK, wout), lambda i: (i, 0))] * 2,
        out_shape=[jax.ShapeDtypeStruct((_N, wout), _f32)] * 2,
    )(agg_a, agg_b, cnt_a, cnt_b, r, bl, wlT, wrT)


def _tc_final(agg_a, agg_b, cnt_a, cnt_b, r, bl, wcT, bcp):
    win = agg_a.shape[1]

    def body(aA, aB, cA, cB, r_r, bl_r, wc_r, bc_r, o_r):
        cnt = cA[...][:, :1] + cB[...][:, :1]
        inv = 1.0 / jnp.maximum(cnt, 1.0)
        h = jnp.maximum((aA[...] + aB[...]) * inv + bl_r[...] + r_r[...], 0.0)
        o_r[...] = jnp.dot(h, wc_r[...], preferred_element_type=_f32) + bc_r[...]

    return pl.pallas_call(
        body, grid=(_GRID,),
        in_specs=[pl.BlockSpec((_BLK, win), lambda i: (i, 0)),
                  pl.BlockSpec((_BLK, win), lambda i: (i, 0)),
                  pl.BlockSpec((_BLK, _CW), lambda i: (i, 0)),
                  pl.BlockSpec((_BLK, _CW), lambda i: (i, 0)),
                  pl.BlockSpec((_BLK, win), lambda i: (i, 0)),
                  pl.BlockSpec((1, win), lambda i: (0, 0)),
                  pl.BlockSpec((win, 128), lambda i: (0, 0)),
                  pl.BlockSpec((1, 128), lambda i: (0, 0))],
        out_specs=pl.BlockSpec((_BLK, 128), lambda i: (i, 0)),
        out_shape=jax.ShapeDtypeStruct((_N, 128), _f32),
    )(agg_a, agg_b, cnt_a, cnt_b, r, bl, wcT, bcp)


def kernel(x, edge_index, Wl1, bl1, Wr1, Wl2, bl2, Wr2, Wl3, bl3, Wr3, Wc, bc):
    src = edge_index[0]
    dst = edge_index[1]
    pad = _EPAD - _E
    src_p = jnp.concatenate(
        [src, jnp.zeros((pad,), jnp.int32)]).reshape(_NW, _CPT, _K)
    dst_p = jnp.concatenate(
        [dst, jnp.full((pad,), _N, jnp.int32)]).reshape(_NW, _CPT, _K)
    z128 = jnp.zeros((_RPT, 128), _f32)
    z64 = jnp.zeros((_RPT, 64), _f32)
    zc = jnp.zeros((_RPT, _CW), _f32)
    ones = jnp.ones((_K, _CW), _f32)
    bl1r = bl1.reshape(1, -1)
    bl2r = bl2.reshape(1, -1)
    bl3r = bl3.reshape(1, -1)
    wcT = jnp.zeros((64, 128), _f32).at[:, :7].set(Wc.T)
    bcp = jnp.zeros((1, 128), _f32).at[0, :7].set(bc)

    cnt_a, cnt_b = _sc_cnt(dst_p, zc, ones)
    y1, r1 = _tc_first(x, Wl1.T, Wr1.T)
    agg_a1, agg_b1 = _sc128(y1, src_p, dst_p, z128)
    y2, r2 = _tc_mid(agg_a1, agg_b1, cnt_a, cnt_b, r1, bl1r, Wl2.T, Wr2.T)
    agg_a2, agg_b2 = _sc128(y2, src_p, dst_p, z128)
    y3, r3 = _tc_mid(agg_a2, agg_b2, cnt_a, cnt_b, r2, bl2r, Wl3.T, Wr3.T)
    agg_a3, agg_b3 = _sc64(y3, src_p, dst_p, z64)
    logits = _tc_final(agg_a3, agg_b3, cnt_a, cnt_b, r3, bl3r, wcT, bcp)
    return logits[:, :7]


# speedup vs baseline: 1.0484x; 1.0484x over previous
import jax
import jax.numpy as jnp
from jax import lax
from jax.experimental import pallas as pl
from jax.experimental.pallas import tpu as pltpu
from jax.experimental.pallas import tpu_sc as plsc

_N = 10000
_E = 320000
_NC, _NS = 2, 16
_NW = _NC * _NS
_K = 64
_CPT = 160
_CG = 32
_NG = _CPT // _CG
_EPAD = _NW * _CPT * _K
_NPAD = 10112
_RPT = _NPAD // _NS
_CW = 16
_BLK = 1000
_GRID = _N // _BLK

_f32 = jnp.float32
_sc_params = pltpu.CompilerParams(use_tc_tiling_on_sc=False)


def _core_outs(c, sl, srcs, outs_a, outs_b):
    @pl.when(c == 0)
    def _():
        for src, out in zip(srcs, outs_a):
            pltpu.sync_copy(src.at[sl], out.at[sl])

    @pl.when(c == 1)
    def _():
        for src, out in zip(srcs, outs_b):
            pltpu.sync_copy(src.at[sl], out.at[sl])


def _make_sc_segsum(width):
    mesh = plsc.VectorSubcoreMesh(core_axis_name="c", subcore_axis_name="s",
                                  num_cores=_NC, num_subcores=_NS)
    out_type = [jax.ShapeDtypeStruct((_NPAD, width), _f32)] * 2
    scratch = [pltpu.VMEM_SHARED((_NPAD, width), _f32),
               pltpu.VMEM((_CG, _K), jnp.int32),
               pltpu.VMEM((_CG, _K), jnp.int32),
               pltpu.VMEM((4, _K, width), _f32),
               pltpu.SemaphoreType.DMA,
               pltpu.SemaphoreType.DMA]

    def body(table, src_hbm, dst_hbm, zeros_hbm, out_a, out_b,
             acc, src_v, dst_v, rows, sem, sem_s):
        c = lax.axis_index("c")
        s = lax.axis_index("s")
        wid = s * _NC + c
        sl = pl.ds(s * _RPT, _RPT)
        pltpu.sync_copy(zeros_hbm, acc.at[sl])
        plsc.subcore_barrier()

        def group(g, carry):
            pltpu.sync_copy(src_hbm.at[wid, pl.ds(g * _CG, _CG)], src_v)
            pltpu.sync_copy(dst_hbm.at[wid, pl.ds(g * _CG, _CG)], dst_v)
            pltpu.async_copy(table.at[src_v.at[0]], rows.at[0], sem)
            pltpu.async_copy(table.at[src_v.at[1]], rows.at[1], sem)

            def chunk(j, carry2):
                b = lax.rem(j, 4)
                bn = lax.rem(j + 2, 4)
                pltpu.make_async_copy(
                    table.at[src_v.at[j]], rows.at[b], sem).wait()

                @pl.when(j >= 2)
                def _():
                    pltpu.make_async_copy(
                        rows.at[bn], acc.at[dst_v.at[j - 2]],
                        sem_s).wait()

                @pl.when(j + 2 < _CG)
                def _():
                    pltpu.async_copy(
                        table.at[src_v.at[j + 2]], rows.at[bn], sem)

                pltpu.async_copy(rows.at[b], acc.at[dst_v.at[j]], sem_s,
                                 add=True)
                return carry2

            lax.fori_loop(0, _CG, chunk, 0)
            pltpu.make_async_copy(
                rows.at[(_CG - 2) % 4], acc.at[dst_v.at[_CG - 2]],
                sem_s).wait()
            pltpu.make_async_copy(
                rows.at[(_CG - 1) % 4], acc.at[dst_v.at[_CG - 1]],
                sem_s).wait()
            return carry

        lax.fori_loop(0, _NG, group, 0)
        plsc.subcore_barrier()
        _core_outs(c, sl, [acc], [out_a], [out_b])

    return pl.kernel(body, out_type, mesh=mesh, scratch_types=scratch,
                     compiler_params=_sc_params)


def _make_sc_count():
    mesh = plsc.VectorSubcoreMesh(core_axis_name="c", subcore_axis_name="s",
                                  num_cores=_NC, num_subcores=_NS)
    out_type = [jax.ShapeDtypeStruct((_NPAD, _CW), _f32)] * 2
    scratch = [pltpu.VMEM_SHARED((_NPAD, _CW), _f32),
               pltpu.VMEM((_CG, _K), jnp.int32),
               pltpu.VMEM((_K, _CW), _f32),
               pltpu.SemaphoreType.DMA]

    def body(dst_hbm, zc_hbm, ones_hbm, cout_a, cout_b,
             cacc, dst_v, ones_v, sem):
        c = lax.axis_index("c")
        s = lax.axis_index("s")
        wid = s * _NC + c
        sl = pl.ds(s * _RPT, _RPT)
        pltpu.sync_copy(zc_hbm, cacc.at[sl])
        pltpu.sync_copy(ones_hbm, ones_v)
        plsc.subcore_barrier()

        def group(g, carry):
            pltpu.sync_copy(dst_hbm.at[wid, pl.ds(g * _CG, _CG)], dst_v)

            def chunk(j, carry2):
                pltpu.async_copy(ones_v, cacc.at[dst_v.at[j]], sem, add=True)

                @pl.when(j > 0)
                def _():
                    pltpu.make_async_copy(
                        ones_v, cacc.at[dst_v.at[j - 1]], sem).wait()

                return carry2

            lax.fori_loop(0, _CG, chunk, 0)
            pltpu.make_async_copy(ones_v, cacc.at[dst_v.at[_CG - 1]],
                                  sem).wait()
            return carry

        lax.fori_loop(0, _NG, group, 0)
        plsc.subcore_barrier()
        _core_outs(c, sl, [cacc], [cout_a], [cout_b])

    return pl.kernel(body, out_type, mesh=mesh, scratch_types=scratch,
                     compiler_params=_sc_params)


_sc128 = _make_sc_segsum(128)
_sc64 = _make_sc_segsum(64)
_sc_cnt = _make_sc_count()


def _tc_first(x, wlT, wrT):
    def body(x_r, wl_r, wr_r, y_r, r_r):
        xb = x_r[...]
        y_r[...] = jnp.dot(xb, wl_r[...], preferred_element_type=_f32)
        r_r[...] = jnp.dot(xb, wr_r[...], preferred_element_type=_f32)

    return pl.pallas_call(
        body, grid=(_GRID,),
        in_specs=[pl.BlockSpec((_BLK, 128), lambda i: (i, 0)),
                  pl.BlockSpec((128, 128), lambda i: (0, 0)),
                  pl.BlockSpec((128, 128), lambda i: (0, 0))],
        out_specs=[pl.BlockSpec((_BLK, 128), lambda i: (i, 0))] * 2,
        out_shape=[jax.ShapeDtypeStruct((_N, 128), _f32)] * 2,
    )(x, wlT, wrT)


def _tc_mid(agg_a, agg_b, cnt_a, cnt_b, r, bl, wlT, wrT):
    win = agg_a.shape[1]
    wout = wlT.shape[1]

    def body(aA, aB, cA, cB, r_r, bl_r, wl_r, wr_r, y_r, rr_r):
        cnt = cA[...][:, :1] + cB[...][:, :1]
        inv = 1.0 / jnp.maximum(cnt, 1.0)
        h = jnp.maximum((aA[...] + aB[...]) * inv + bl_r[...] + r_r[...], 0.0)
        y_r[...] = jnp.dot(h, wl_r[...], preferred_element_type=_f32)
        rr_r[...] = jnp.dot(h, wr_r[...], preferred_element_type=_f32)

    return pl.pallas_call(
        body, grid=(_GRID,),
        in_specs=[pl.BlockSpec((_BLK, win), lambda i: (i, 0)),
                  pl.BlockSpec((_BLK, win), lambda i: (i, 0)),
                  pl.BlockSpec((_BLK, _CW), lambda i: (i, 0)),
                  pl.BlockSpec((_BLK, _CW), lambda i: (i, 0)),
                  pl.BlockSpec((_BLK, win), lambda i: (i, 0)),
                  pl.BlockSpec((1, win), lambda i: (0, 0)),
                  pl.BlockSpec((win, wout), lambda i: (0, 0)),
                  pl.BlockSpec((win, wout), lambda i: (0, 0))],
        out_specs=[pl.BlockSpec((_BLK, wout), lambda i: (i, 0))] * 2,
        out_shape=[jax.ShapeDtypeStruct((_N, wout), _f32)] * 2,
    )(agg_a, agg_b, cnt_a, cnt_b, r, bl, wlT, wrT)


def _tc_final(agg_a, agg_b, cnt_a, cnt_b, r, bl, wcT, bcp):
    win = agg_a.shape[1]

    def body(aA, aB, cA, cB, r_r, bl_r, wc_r, bc_r, o_r):
        cnt = cA[...][:, :1] + cB[...][:, :1]
        inv = 1.0 / jnp.maximum(cnt, 1.0)
        h = jnp.maximum((aA[...] + aB[...]) * inv + bl_r[...] + r_r[...], 0.0)
        o_r[...] = jnp.dot(h, wc_r[...], preferred_element_type=_f32) + bc_r[...]

    return pl.pallas_call(
        body, grid=(_GRID,),
        in_specs=[pl.BlockSpec((_BLK, win), lambda i: (i, 0)),
                  pl.BlockSpec((_BLK, win), lambda i: (i, 0)),
                  pl.BlockSpec((_BLK, _CW), lambda i: (i, 0)),
                  pl.BlockSpec((_BLK, _CW), lambda i: (i, 0)),
                  pl.BlockSpec((_BLK, win), lambda i: (i, 0)),
                  pl.BlockSpec((1, win), lambda i: (0, 0)),
                  pl.BlockSpec((win, 128), lambda i: (0, 0)),
                  pl.BlockSpec((1, 128), lambda i: (0, 0))],
        out_specs=pl.BlockSpec((_BLK, 128), lambda i: (i, 0)),
        out_shape=jax.ShapeDtypeStruct((_N, 128), _f32),
    )(agg_a, agg_b, cnt_a, cnt_b, r, bl, wcT, bcp)


def kernel(x, edge_index, Wl1, bl1, Wr1, Wl2, bl2, Wr2, Wl3, bl3, Wr3, Wc, bc):
    src = edge_index[0]
    dst = edge_index[1]
    pad = _EPAD - _E
    src_p = jnp.concatenate(
        [src, jnp.zeros((pad,), jnp.int32)]).reshape(_NW, _CPT, _K)
    dst_p = jnp.concatenate(
        [dst, jnp.full((pad,), _N, jnp.int32)]).reshape(_NW, _CPT, _K)
    z128 = jnp.zeros((_RPT, 128), _f32)
    z64 = jnp.zeros((_RPT, 64), _f32)
    zc = jnp.zeros((_RPT, _CW), _f32)
    ones = jnp.ones((_K, _CW), _f32)
    bl1r = bl1.reshape(1, -1)
    bl2r = bl2.reshape(1, -1)
    bl3r = bl3.reshape(1, -1)
    wcT = jnp.zeros((64, 128), _f32).at[:, :7].set(Wc.T)
    bcp = jnp.zeros((1, 128), _f32).at[0, :7].set(bc)

    cnt_a, cnt_b = _sc_cnt(dst_p, zc, ones)
    y1, r1 = _tc_first(x, Wl1.T, Wr1.T)
    agg_a1, agg_b1 = _sc128(y1, src_p, dst_p, z128)
    y2, r2 = _tc_mid(agg_a1, agg_b1, cnt_a, cnt_b, r1, bl1r, Wl2.T, Wr2.T)
    agg_a2, agg_b2 = _sc128(y2, src_p, dst_p, z128)
    y3, r3 = _tc_mid(agg_a2, agg_b2, cnt_a, cnt_b, r2, bl2r, Wl3.T, Wr3.T)
    agg_a3, agg_b3 = _sc64(y3, src_p, dst_p, z64)
    logits = _tc_final(agg_a3, agg_b3, cnt_a, cnt_b, r3, bl3r, wcT, bcp)
    return logits[:, :7]
